# baseline (speedup 1.0000x reference)
import functools

import jax
import jax.numpy as jnp
from jax import lax
from jax.experimental import pallas as pl
from jax.experimental.pallas import tpu as pltpu
from jax.experimental.pallas import tpu_sc as plsc

N = 10000
D = 128
E = 320000
L = 4
EPS = 1e-5

NC = 2
NS = 16
NW = NC * NS
K = 128
CHUNKS = 80
WE = CHUNKS * K
EP = NW * WE
RPT = 632
N_ACC = NS * RPT

_MESH = plsc.VectorSubcoreMesh(core_axis_name="c", subcore_axis_name="s")

f32 = jnp.float32


def _seg_body(with_gather, *refs):
    if with_gather:
        (z_hbm, row_hbm, col_hbm, zD_hbm, out_hbm, acc_sh,
         cb0, cb1, rb0, rb1, rb2, rb3, rv0, rv1,
         gs0, gs1, ss0, ss1, ic0, ic1, ir0, ir1, ir2, ir3) = refs
        cbs = (cb0, cb1)
        rvs = (rv0, rv1)
        gss = (gs0, gs1)
        ics = (ic0, ic1)
    else:
        (row_hbm, ones_hbm, zD_hbm, out_hbm, acc_sh,
         rb0, rb1, rb2, rb3, ones_v,
         ss0, ss1, ir0, ir1, ir2, ir3) = refs
    rbs = (rb0, rb1, rb2, rb3)
    sss = (ss0, ss1)
    irs = (ir0, ir1, ir2, ir3)

    cid = lax.axis_index("c")
    sid = lax.axis_index("s")
    wid = sid * NC + cid
    r0 = sid * RPT
    ebase = wid * WE

    pltpu.sync_copy(zD_hbm.at[pl.ds(r0, RPT)], acc_sh.at[pl.ds(r0, RPT)])
    if not with_gather:
        pltpu.sync_copy(ones_hbm, ones_v)
    plsc.subcore_barrier()

    def fire_ridx(c, q):
        pltpu.async_copy(row_hbm.at[pl.ds(ebase + c * K, K)], rbs[q], irs[q])

    def wait_ridx(q):
        pltpu.make_async_copy(row_hbm.at[pl.ds(ebase, K)], rbs[q],
                              irs[q]).wait()

    def fire_cidx(c, b):
        pltpu.async_copy(col_hbm.at[pl.ds(ebase + c * K, K)], cbs[b], ics[b])

    def wait_cidx(b):
        pltpu.make_async_copy(col_hbm.at[pl.ds(ebase, K)], cbs[b],
                              ics[b]).wait()

    def fire_gather(b):
        pltpu.async_copy(z_hbm.at[cbs[b]], rvs[b], gss[b])

    def wait_gather(b):
        pltpu.make_async_copy(z_hbm.at[cbs[b]], rvs[b], gss[b]).wait()

    def fire_scatter(b, q):
        src = rvs[b] if with_gather else ones_v
        pltpu.async_copy(src, acc_sh.at[rbs[q]], sss[b], add=True)

    def wait_scatter(b):
        src = rvs[b] if with_gather else ones_v
        pltpu.make_async_copy(src, acc_sh.at[rbs[0]], sss[b]).wait()

    def chunk(c, q, fire_next_idx=True, fire_next_gather=True,
              wait_prev_scatter=True):
        b = q & 1
        if with_gather:
            wait_gather(b)
            wait_ridx(q)
        else:
            wait_ridx(q)
            if wait_prev_scatter:
                wait_scatter(b)
        fire_scatter(b, q)
        if fire_next_idx:
            if with_gather:
                fire_cidx(c + 2, b)
            fire_ridx(c + 2, (q + 2) & 3)
        if with_gather and fire_next_gather:
            wait_cidx(1 - b)
            if wait_prev_scatter:
                wait_scatter(1 - b)
            fire_gather(1 - b)

    if with_gather:
        fire_cidx(0, 0)
        fire_cidx(1, 1)
    fire_ridx(0, 0)
    fire_ridx(1, 1)
    if with_gather:
        wait_cidx(0)
        fire_gather(0)

    chunk(0, 0, wait_prev_scatter=False)
    chunk(1, 1, wait_prev_scatter=with_gather)
    chunk(2, 2, wait_prev_scatter=True)
    chunk(3, 3, wait_prev_scatter=True)

    def quad(i, carry):
        c0 = i * 4
        for u in range(4):
            chunk(c0 + u, u)
        return carry
    lax.fori_loop(1, (CHUNKS - 8) // 4 + 1, quad, 0)

    chunk(CHUNKS - 4, 0)
    chunk(CHUNKS - 3, 1)
    chunk(CHUNKS - 2, 2, fire_next_idx=False)
    chunk(CHUNKS - 1, 3, fire_next_idx=False, fire_next_gather=False)
    wait_scatter(0)
    wait_scatter(1)

    plsc.subcore_barrier()
    pltpu.sync_copy(acc_sh.at[pl.ds(r0, RPT)],
                    out_hbm.at[pl.ds(cid * N_ACC + r0, RPT)])


_seg_call = pl.kernel(
    functools.partial(_seg_body, True),
    out_type=jax.ShapeDtypeStruct((NC * N_ACC, D), f32),
    mesh=_MESH,
    scratch_types=[
        pltpu.VMEM_SHARED((N_ACC, D), f32),
        pltpu.VMEM((K,), jnp.int32),
        pltpu.VMEM((K,), jnp.int32),
        pltpu.VMEM((K,), jnp.int32),
        pltpu.VMEM((K,), jnp.int32),
        pltpu.VMEM((K,), jnp.int32),
        pltpu.VMEM((K,), jnp.int32),
        pltpu.VMEM((K, D), f32),
        pltpu.VMEM((K, D), f32),
    ] + [pltpu.SemaphoreType.DMA] * 10,
)

_deg_call = pl.kernel(
    functools.partial(_seg_body, False),
    out_type=jax.ShapeDtypeStruct((NC * N_ACC, D), f32),
    mesh=_MESH,
    scratch_types=[
        pltpu.VMEM_SHARED((N_ACC, D), f32),
        pltpu.VMEM((K,), jnp.int32),
        pltpu.VMEM((K,), jnp.int32),
        pltpu.VMEM((K,), jnp.int32),
        pltpu.VMEM((K,), jnp.int32),
        pltpu.VMEM((K, D), f32),
    ] + [pltpu.SemaphoreType.DMA] * 6,
)


RB = 2000
GRID = N // RB


def _full(shape):
    nd = len(shape)
    return pl.BlockSpec(shape, lambda r: (0,) * nd)


def _rows(width=D):
    return pl.BlockSpec((RB, width), lambda r: (r, 0))


def _dot(a, b):
    return jnp.dot(a, b, preferred_element_type=f32)


def _pre_body(x_ref, W1, b1, s1, be, W2, b2, Wp, bp, Wn0, Ws0, bs0,
              h_ref, z_ref, lin_ref):
    t = _dot(x_ref[...], W1[...]) + b1[...]
    t = jnp.maximum(t * s1[...] + be[...], 0.0)
    t = _dot(t, W2[...]) + b2[...]
    h = _dot(t, Wp[...]) + bp[...]
    h_ref[...] = h
    z_ref[...] = _dot(h, Wn0[...])
    lin_ref[...] = _dot(h, Ws0[...]) + bs0[...]


_pre_call = pl.pallas_call(
    _pre_body,
    grid=(GRID,),
    in_specs=[_rows(), _full((D, D)), _full((1, D)), _full((1, D)),
              _full((1, D)), _full((D, D)), _full((1, D)), _full((D, D)),
              _full((1, D)), _full((D, D)), _full((D, D)), _full((1, D))],
    out_specs=[_rows(), _rows(), _rows()],
    out_shape=[jax.ShapeDtypeStruct((N, D), f32)] * 3,
)


def _mid_body(first, *refs):
    if first:
        (h_ref, part_ref, degp_ref, lin_ref, bnb, scl, bet, Wnn, Wsn, bsn,
         hn_ref, zn_ref, linn_ref, inv_ref) = refs
        degs = degp_ref[...]
        deg = jnp.maximum((degs[0] + degs[1])[:, 0:1], 1.0)
        inv = 1.0 / deg
        inv_ref[...] = inv
    else:
        (h_ref, part_ref, inv_ref, lin_ref, bnb, scl, bet, Wnn, Wsn, bsn,
         hn_ref, zn_ref, linn_ref) = refs
        inv = inv_ref[...]
    part = part_ref[...]
    s = part[0] + part[1]
    o = lin_ref[...] + s * inv + bnb[...]
    o = jnp.maximum(o * scl[...] + bet[...], 0.0)
    hn = h_ref[...] + o
    hn_ref[...] = hn
    zn_ref[...] = _dot(hn, Wnn[...])
    linn_ref[...] = _dot(hn, Wsn[...]) + bsn[...]


_part_spec = pl.BlockSpec((NC, RB, D), lambda r: (0, r, 0))
_degp_spec = pl.BlockSpec((NC, RB, D), lambda r: (0, r, 0))
_inv_spec = pl.BlockSpec((RB, 1), lambda r: (r, 0))

_mid0_call = pl.pallas_call(
    functools.partial(_mid_body, True),
    grid=(GRID,),
    in_specs=[_rows(), _part_spec, _degp_spec, _rows(), _full((1, D)),
              _full((1, D)), _full((1, D)), _full((D, D)), _full((D, D)),
              _full((1, D))],
    out_specs=[_rows(), _rows(), _rows(), _inv_spec],
    out_shape=[jax.ShapeDtypeStruct((N, D), f32)] * 3
    + [jax.ShapeDtypeStruct((N, 1), f32)],
)

_mid_call = pl.pallas_call(
    functools.partial(_mid_body, False),
    grid=(GRID,),
    in_specs=[_rows(), _part_spec, _inv_spec, _rows(), _full((1, D)),
              _full((1, D)), _full((1, D)), _full((D, D)), _full((D, D)),
              _full((1, D))],
    out_specs=[_rows(), _rows(), _rows()],
    out_shape=[jax.ShapeDtypeStruct((N, D), f32)] * 3,
)


def _fin_body(h_ref, part_ref, inv_ref, lin_ref, bnb, scl, bet,
              hW1, hb1, hW2, hb2, out_ref):
    part = part_ref[...]
    s = part[0] + part[1]
    o = lin_ref[...] + s * inv_ref[...] + bnb[...]
    o = jnp.maximum(o * scl[...] + bet[...], 0.0)
    hf = h_ref[...] + o
    y = jnp.maximum(_dot(hf, hW1[...]) + hb1[...], 0.0)
    out_ref[...] = _dot(y, hW2[...]) + hb2[...]


_fin_call = pl.pallas_call(
    _fin_body,
    grid=(GRID,),
    in_specs=[_rows(), _part_spec, _inv_spec, _rows(), _full((1, D)),
              _full((1, D)), _full((1, D)), _full((D, D // 2)),
              _full((1, D // 2)), _full((D // 2, 1)), _full((1, 1))],
    out_specs=[_inv_spec],
    out_shape=[jax.ShapeDtypeStruct((N, 1), f32)],
)


def kernel(x, edge_index, enc_W1, enc_b1, enc_g, enc_beta, enc_W2, enc_b2,
           proj_W, proj_b, Ws, bs, Wn, bnb, g, beta,
           head_W1, head_b1, head_W2, head_b2):
    row = edge_index[0]
    col = edge_index[1]
    pad = EP - E
    rowp = jnp.concatenate([row, jnp.full((pad,), N, jnp.int32)])
    colp = jnp.concatenate([col, jnp.zeros((pad,), jnp.int32)])
    zerosD = jnp.zeros((N_ACC, D), f32)
    onesKD = jnp.ones((K, D), f32)

    r1 = lambda v: v.reshape(1, -1)
    bn_scale = 1.0 / jnp.sqrt(1.0 + EPS)
    s_enc = r1(enc_g * bn_scale)
    scl = g * bn_scale

    h, z, lin = _pre_call(x, enc_W1, r1(enc_b1), s_enc, r1(enc_beta),
                          enc_W2, r1(enc_b2), proj_W, r1(proj_b),
                          Wn[0], Ws[0], r1(bs[0]))

    part = _seg_call(z, rowp, colp, zerosD).reshape(NC, N_ACC, D)
    degp = _deg_call(rowp, onesKD, zerosD).reshape(NC, N_ACC, D)
    h, z, lin, inv = _mid0_call(h, part, degp, lin, r1(bnb[0]), r1(scl[0]),
                                r1(beta[0]), Wn[1], Ws[1], r1(bs[1]))
    for i in (1, 2):
        part = _seg_call(z, rowp, colp, zerosD).reshape(NC, N_ACC, D)
        h, z, lin = _mid_call(h, part, inv, lin, r1(bnb[i]), r1(scl[i]),
                              r1(beta[i]), Wn[i + 1], Ws[i + 1],
                              r1(bs[i + 1]))
    part = _seg_call(z, rowp, colp, zerosD).reshape(NC, N_ACC, D)
    (out,) = _fin_call(h, part, inv, lin, r1(bnb[3]), r1(scl[3]), r1(beta[3]),
                       head_W1, r1(head_b1), head_W2, head_b2.reshape(1, 1))
    return out[:, 0]


# speedup vs baseline: 1.0408x; 1.0408x over previous
import functools

import jax
import jax.numpy as jnp
from jax import lax
from jax.experimental import pallas as pl
from jax.experimental.pallas import tpu as pltpu
from jax.experimental.pallas import tpu_sc as plsc

N = 10000
D = 128
E = 320000
L = 4
EPS = 1e-5

NC = 2
NS = 16
NW = NC * NS
WE = 10240
EP = NW * WE
RPT = 632
N_ACC = NS * RPT

KG = 64
CH = WE // KG
G = 4
RING = 8

KD = 128
CHD = WE // KD

_MESH = plsc.VectorSubcoreMesh(core_axis_name="c", subcore_axis_name="s")

f32 = jnp.float32


def _gseg_body(z_hbm, row_hbm, col_hbm, zD_hbm, out_hbm, acc_sh, *bufs):
    cbs = bufs[0:RING]
    rbs = bufs[RING:2 * RING]
    rvs = bufs[2 * RING:2 * RING + G]
    gss = bufs[2 * RING + G:2 * RING + 2 * G]
    sss = bufs[2 * RING + 2 * G:2 * RING + 3 * G]
    ics = bufs[2 * RING + 3 * G:3 * RING + 3 * G]
    irs = bufs[3 * RING + 3 * G:4 * RING + 3 * G]

    cid = lax.axis_index("c")
    sid = lax.axis_index("s")
    wid = sid * NC + cid
    r0 = sid * RPT
    ebase = wid * WE

    pltpu.sync_copy(zD_hbm.at[pl.ds(r0, RPT)], acc_sh.at[pl.ds(r0, RPT)])
    plsc.subcore_barrier()

    def fire_idx(c, s):
        pltpu.async_copy(col_hbm.at[pl.ds(ebase + c * KG, KG)], cbs[s],
                         ics[s])
        pltpu.async_copy(row_hbm.at[pl.ds(ebase + c * KG, KG)], rbs[s],
                         irs[s])

    def fire_gather(s, q):
        pltpu.async_copy(z_hbm.at[cbs[s]], rvs[q], gss[q])

    def chunk(c, u, wait_prev_scatter=True, do_idx=True, do_gather=True):
        q = u & 3
        r = u & 7
        rn = (u + 7) & 7
        qg = (u + 3) & 3
        rg = (u + 3) & 7
        pltpu.make_async_copy(z_hbm.at[cbs[r]], rvs[q], gss[q]).wait()
        pltpu.make_async_copy(row_hbm.at[pl.ds(ebase, KG)], rbs[r],
                              irs[r]).wait()
        pltpu.async_copy(rvs[q], acc_sh.at[rbs[r]], sss[q], add=True)
        if wait_prev_scatter:
            pltpu.make_async_copy(rvs[qg], acc_sh.at[rbs[0]], sss[qg]).wait()
        if do_idx:
            fire_idx(c + 7, rn)
        if do_gather:
            pltpu.make_async_copy(col_hbm.at[pl.ds(ebase, KG)], cbs[rg],
                                  ics[rg]).wait()
            fire_gather(rg, qg)

    for c in range(7):
        fire_idx(c, c)
    for c in range(3):
        pltpu.make_async_copy(col_hbm.at[pl.ds(ebase, KG)], cbs[c],
                              ics[c]).wait()
        fire_gather(c, c)

    chunk(0, 0, wait_prev_scatter=False)
    for c in range(1, 8):
        chunk(c, c)

    def octet(i, carry):
        c0 = i * 8
        for u in range(8):
            chunk(c0 + u, u)
        return carry
    lax.fori_loop(1, CH // 8 - 1, octet, 0)

    for c in range(CH - 8, CH):
        u = c & 7
        chunk(c, u, do_idx=(c + 7 < CH), do_gather=(c + 3 < CH))
    pltpu.make_async_copy(rvs[(CH - 1) & 3], acc_sh.at[rbs[0]],
                          sss[(CH - 1) & 3]).wait()

    plsc.subcore_barrier()
    pltpu.sync_copy(acc_sh.at[pl.ds(r0, RPT)],
                    out_hbm.at[pl.ds(cid * N_ACC + r0, RPT)])


def _deg_body(row_hbm, ones_hbm, zD_hbm, out_hbm, acc_sh,
              rb0, rb1, rb2, rb3, ones_v, ss0, ss1, ir0, ir1, ir2, ir3):
    rbs = (rb0, rb1, rb2, rb3)
    sss = (ss0, ss1)
    irs = (ir0, ir1, ir2, ir3)

    cid = lax.axis_index("c")
    sid = lax.axis_index("s")
    wid = sid * NC + cid
    r0 = sid * RPT
    ebase = wid * WE

    pltpu.sync_copy(zD_hbm.at[pl.ds(r0, RPT)], acc_sh.at[pl.ds(r0, RPT)])
    pltpu.sync_copy(ones_hbm, ones_v)
    plsc.subcore_barrier()

    def fire_ridx(c, q):
        pltpu.async_copy(row_hbm.at[pl.ds(ebase + c * KD, KD)], rbs[q],
                         irs[q])

    def chunk(c, q, wait_prev_scatter=True, fire_next_idx=True):
        b = q & 1
        pltpu.make_async_copy(row_hbm.at[pl.ds(ebase, KD)], rbs[q],
                              irs[q]).wait()
        if wait_prev_scatter:
            pltpu.make_async_copy(ones_v, acc_sh.at[rbs[0]], sss[b]).wait()
        pltpu.async_copy(ones_v, acc_sh.at[rbs[q]], sss[b], add=True)
        if fire_next_idx:
            fire_ridx(c + 2, (q + 2) & 3)

    fire_ridx(0, 0)
    fire_ridx(1, 1)
    chunk(0, 0, wait_prev_scatter=False)
    chunk(1, 1, wait_prev_scatter=False)
    chunk(2, 2)
    chunk(3, 3)

    def quad(i, carry):
        c0 = i * 4
        for u in range(4):
            chunk(c0 + u, u)
        return carry
    lax.fori_loop(1, CHD // 4 - 1, quad, 0)

    chunk(CHD - 4, 0)
    chunk(CHD - 3, 1)
    chunk(CHD - 2, 2, fire_next_idx=False)
    chunk(CHD - 1, 3, fire_next_idx=False)
    pltpu.make_async_copy(ones_v, acc_sh.at[rbs[0]], sss[0]).wait()
    pltpu.make_async_copy(ones_v, acc_sh.at[rbs[0]], sss[1]).wait()

    plsc.subcore_barrier()
    pltpu.sync_copy(acc_sh.at[pl.ds(r0, RPT)],
                    out_hbm.at[pl.ds(cid * N_ACC + r0, RPT)])


_seg_call = pl.kernel(
    _gseg_body,
    out_type=jax.ShapeDtypeStruct((NC * N_ACC, D), f32),
    mesh=_MESH,
    scratch_types=(
        [pltpu.VMEM_SHARED((N_ACC, D), f32)]
        + [pltpu.VMEM((KG,), jnp.int32)] * RING
        + [pltpu.VMEM((KG,), jnp.int32)] * RING
        + [pltpu.VMEM((KG, D), f32)] * G
        + [pltpu.SemaphoreType.DMA] * (3 * G + 2 * RING)
    ),
)

_deg_call = pl.kernel(
    _deg_body,
    out_type=jax.ShapeDtypeStruct((NC * N_ACC, D), f32),
    mesh=_MESH,
    scratch_types=(
        [pltpu.VMEM_SHARED((N_ACC, D), f32)]
        + [pltpu.VMEM((KD,), jnp.int32)] * 4
        + [pltpu.VMEM((KD, D), f32)]
        + [pltpu.SemaphoreType.DMA] * 6
    ),
)


RB = 2000
GRID = N // RB


def _full(shape):
    nd = len(shape)
    return pl.BlockSpec(shape, lambda r: (0,) * nd)


def _rows(width=D):
    return pl.BlockSpec((RB, width), lambda r: (r, 0))


def _dot(a, b):
    return jnp.dot(a, b, preferred_element_type=f32)


def _pre_body(x_ref, W1, b1, s1, be, W2, b2, Wp, bp, Wn0, Ws0, bs0,
              h_ref, z_ref, lin_ref):
    t = _dot(x_ref[...], W1[...]) + b1[...]
    t = jnp.maximum(t * s1[...] + be[...], 0.0)
    t = _dot(t, W2[...]) + b2[...]
    h = _dot(t, Wp[...]) + bp[...]
    h_ref[...] = h
    z_ref[...] = _dot(h, Wn0[...])
    lin_ref[...] = _dot(h, Ws0[...]) + bs0[...]


_pre_call = pl.pallas_call(
    _pre_body,
    grid=(GRID,),
    in_specs=[_rows(), _full((D, D)), _full((1, D)), _full((1, D)),
              _full((1, D)), _full((D, D)), _full((1, D)), _full((D, D)),
              _full((1, D)), _full((D, D)), _full((D, D)), _full((1, D))],
    out_specs=[_rows(), _rows(), _rows()],
    out_shape=[jax.ShapeDtypeStruct((N, D), f32)] * 3,
)


def _mid_body(first, *refs):
    if first:
        (h_ref, part_ref, degp_ref, lin_ref, bnb, scl, bet, Wnn, Wsn, bsn,
         hn_ref, zn_ref, linn_ref, inv_ref) = refs
        degs = degp_ref[...]
        deg = jnp.maximum((degs[0] + degs[1])[:, 0:1], 1.0)
        inv = 1.0 / deg
        inv_ref[...] = inv
    else:
        (h_ref, part_ref, inv_ref, lin_ref, bnb, scl, bet, Wnn, Wsn, bsn,
         hn_ref, zn_ref, linn_ref) = refs
        inv = inv_ref[...]
    part = part_ref[...]
    s = part[0] + part[1]
    o = lin_ref[...] + s * inv + bnb[...]
    o = jnp.maximum(o * scl[...] + bet[...], 0.0)
    hn = h_ref[...] + o
    hn_ref[...] = hn
    zn_ref[...] = _dot(hn, Wnn[...])
    linn_ref[...] = _dot(hn, Wsn[...]) + bsn[...]


_part_spec = pl.BlockSpec((NC, RB, D), lambda r: (0, r, 0))
_degp_spec = pl.BlockSpec((NC, RB, D), lambda r: (0, r, 0))
_inv_spec = pl.BlockSpec((RB, 1), lambda r: (r, 0))

_mid0_call = pl.pallas_call(
    functools.partial(_mid_body, True),
    grid=(GRID,),
    in_specs=[_rows(), _part_spec, _degp_spec, _rows(), _full((1, D)),
              _full((1, D)), _full((1, D)), _full((D, D)), _full((D, D)),
              _full((1, D))],
    out_specs=[_rows(), _rows(), _rows(), _inv_spec],
    out_shape=[jax.ShapeDtypeStruct((N, D), f32)] * 3
    + [jax.ShapeDtypeStruct((N, 1), f32)],
)

_mid_call = pl.pallas_call(
    functools.partial(_mid_body, False),
    grid=(GRID,),
    in_specs=[_rows(), _part_spec, _inv_spec, _rows(), _full((1, D)),
              _full((1, D)), _full((1, D)), _full((D, D)), _full((D, D)),
              _full((1, D))],
    out_specs=[_rows(), _rows(), _rows()],
    out_shape=[jax.ShapeDtypeStruct((N, D), f32)] * 3,
)


def _fin_body(h_ref, part_ref, inv_ref, lin_ref, bnb, scl, bet,
              hW1, hb1, hW2, hb2, out_ref):
    part = part_ref[...]
    s = part[0] + part[1]
    o = lin_ref[...] + s * inv_ref[...] + bnb[...]
    o = jnp.maximum(o * scl[...] + bet[...], 0.0)
    hf = h_ref[...] + o
    y = jnp.maximum(_dot(hf, hW1[...]) + hb1[...], 0.0)
    out_ref[...] = _dot(y, hW2[...]) + hb2[...]


_fin_call = pl.pallas_call(
    _fin_body,
    grid=(GRID,),
    in_specs=[_rows(), _part_spec, _inv_spec, _rows(), _full((1, D)),
              _full((1, D)), _full((1, D)), _full((D, D // 2)),
              _full((1, D // 2)), _full((D // 2, 1)), _full((1, 1))],
    out_specs=[_inv_spec],
    out_shape=[jax.ShapeDtypeStruct((N, 1), f32)],
)


def kernel(x, edge_index, enc_W1, enc_b1, enc_g, enc_beta, enc_W2, enc_b2,
           proj_W, proj_b, Ws, bs, Wn, bnb, g, beta,
           head_W1, head_b1, head_W2, head_b2):
    row = edge_index[0]
    col = edge_index[1]
    pad = EP - E
    rowp = jnp.concatenate([row, jnp.full((pad,), N, jnp.int32)])
    colp = jnp.concatenate([col, jnp.zeros((pad,), jnp.int32)])
    zerosD = jnp.zeros((N_ACC, D), f32)
    onesKD = jnp.ones((KD, D), f32)

    r1 = lambda v: v.reshape(1, -1)
    bn_scale = 1.0 / jnp.sqrt(1.0 + EPS)
    s_enc = r1(enc_g * bn_scale)
    scl = g * bn_scale

    h, z, lin = _pre_call(x, enc_W1, r1(enc_b1), s_enc, r1(enc_beta),
                          enc_W2, r1(enc_b2), proj_W, r1(proj_b),
                          Wn[0], Ws[0], r1(bs[0]))

    part = _seg_call(z, rowp, colp, zerosD).reshape(NC, N_ACC, D)
    degp = _deg_call(rowp, onesKD, zerosD).reshape(NC, N_ACC, D)
    h, z, lin, inv = _mid0_call(h, part, degp, lin, r1(bnb[0]), r1(scl[0]),
                                r1(beta[0]), Wn[1], Ws[1], r1(bs[1]))
    for i in (1, 2):
        part = _seg_call(z, rowp, colp, zerosD).reshape(NC, N_ACC, D)
        h, z, lin = _mid_call(h, part, inv, lin, r1(bnb[i]), r1(scl[i]),
                              r1(beta[i]), Wn[i + 1], Ws[i + 1],
                              r1(bs[i + 1]))
    part = _seg_call(z, rowp, colp, zerosD).reshape(NC, N_ACC, D)
    (out,) = _fin_call(h, part, inv, lin, r1(bnb[3]), r1(scl[3]), r1(beta[3]),
                       head_W1, r1(head_b1), head_W2, head_b2.reshape(1, 1))
    return out[:, 0]
